# initial kernel scaffold (unmeasured)
import jax
import jax.numpy as jnp
from jax import lax
from jax.experimental import pallas as pl
from jax.experimental.pallas import tpu as pltpu

N_DEV = 4
M_CHUNK = 1024
N_TOTAL = 8192
TN = 2048
NTILES = N_TOTAL // TN
N_HOPS = N_DEV - 1


def kernel(x, w_mat, scale_x, scale_w):
    def body(x_ref, w_ref, sx_ref, sw_ref, out_ref,
             send_buf, recv_bufs, send_sems, recv_sems):
        my = lax.axis_index("i")
        right = (my + 1) % N_DEV
        left = (my + 3) % N_DEV

        barrier_sem = pltpu.get_barrier_semaphore()
        for nbr in [left, right]:
            pl.semaphore_signal(
                barrier_sem, inc=1,
                device_id=(nbr,), device_id_type=pl.DeviceIdType.MESH,
            )
        pl.semaphore_wait(barrier_sem, 2)

        scale = sx_ref[0] * sw_ref[0]

        def partial(c, n0):
            xa = x_ref[pl.ds(c * M_CHUNK, M_CHUNK), :]
            wb = w_ref[:, n0:n0 + TN]
            return jnp.dot(xa, wb, preferred_element_type=jnp.float32)

        for nt in range(NTILES):
            n0 = nt * TN
            send_buf[...] = partial((my + 3) % N_DEV, n0)
            for s in range(N_HOPS):
                rdma = pltpu.make_async_remote_copy(
                    src_ref=send_buf,
                    dst_ref=recv_bufs.at[s],
                    send_sem=send_sems.at[nt, s],
                    recv_sem=recv_sems.at[nt, s],
                    device_id=(right,),
                    device_id_type=pl.DeviceIdType.MESH,
                )
                rdma.start()
                rdma.wait()
                if s < N_HOPS - 1:
                    send_buf[...] = recv_bufs[s] + partial(
                        (my + 2 - s) % N_DEV, n0)
                else:
                    out_ref[:, n0:n0 + TN] = (
                        recv_bufs[s] + partial(my, n0)) * scale

    return pl.pallas_call(
        body,
        out_shape=jax.ShapeDtypeStruct((M_CHUNK, N_TOTAL), jnp.float32),
        in_specs=[
            pl.BlockSpec(memory_space=pltpu.VMEM),
            pl.BlockSpec(memory_space=pltpu.VMEM),
            pl.BlockSpec(memory_space=pltpu.SMEM),
            pl.BlockSpec(memory_space=pltpu.SMEM),
        ],
        out_specs=pl.BlockSpec(memory_space=pltpu.VMEM),
        scratch_shapes=[
            pltpu.VMEM((M_CHUNK, TN), jnp.float32),
            pltpu.VMEM((N_HOPS, M_CHUNK, TN), jnp.float32),
            pltpu.SemaphoreType.DMA((NTILES, N_HOPS)),
            pltpu.SemaphoreType.DMA((NTILES, N_HOPS)),
        ],
        compiler_params=pltpu.CompilerParams(collective_id=0),
    )(x, w_mat, scale_x, scale_w)


# baseline (device time: 1233402 ns/iter reference)
import jax
import jax.numpy as jnp
from jax import lax
from jax.experimental import pallas as pl
from jax.experimental.pallas import tpu as pltpu

N_DEV = 4
M_CHUNK = 1024
N_TOTAL = 8192
TN = 1024
NTILES = N_TOTAL // TN
N_HOPS = N_DEV - 1


def kernel(x, w_mat, scale_x, scale_w):
    def body(x_ref, w_ref, sx_ref, sw_ref, out_ref,
             send_buf, recv_bufs, send_sems, recv_sems):
        my = lax.axis_index("i")
        right = (my + 1) % N_DEV
        left = (my + 3) % N_DEV
        nt = pl.program_id(0)

        @pl.when(nt == 0)
        def _():
            barrier_sem = pltpu.get_barrier_semaphore()
            for nbr in [left, right]:
                pl.semaphore_signal(
                    barrier_sem, inc=1,
                    device_id=(nbr,), device_id_type=pl.DeviceIdType.MESH,
                )
            pl.semaphore_wait(barrier_sem, 2)

        scale = sx_ref[0] * sw_ref[0]

        def partial(c):
            xa = x_ref[pl.ds(c * M_CHUNK, M_CHUNK), :].astype(jnp.bfloat16)
            wb = w_ref[...].astype(jnp.bfloat16)
            return jnp.dot(xa, wb, preferred_element_type=jnp.float32)

        send_buf[...] = partial((my + 3) % N_DEV)
        for s in range(N_HOPS):
            rdma = pltpu.make_async_remote_copy(
                src_ref=send_buf,
                dst_ref=recv_bufs.at[s],
                send_sem=send_sems.at[nt, s],
                recv_sem=recv_sems.at[nt, s],
                device_id=(right,),
                device_id_type=pl.DeviceIdType.MESH,
            )
            rdma.start()
            rdma.wait()
            if s < N_HOPS - 1:
                send_buf[...] = recv_bufs[s] + partial((my + 2 - s) % N_DEV)
            else:
                out_ref[...] = (recv_bufs[s] + partial(my)) * scale

    grid = (NTILES,)
    return pl.pallas_call(
        body,
        grid=grid,
        out_shape=jax.ShapeDtypeStruct((M_CHUNK, N_TOTAL), jnp.float32),
        in_specs=[
            pl.BlockSpec((4096, 1024), lambda nt: (0, 0)),
            pl.BlockSpec((1024, TN), lambda nt: (0, nt)),
            pl.BlockSpec(memory_space=pltpu.SMEM),
            pl.BlockSpec(memory_space=pltpu.SMEM),
        ],
        out_specs=pl.BlockSpec((M_CHUNK, TN), lambda nt: (0, nt)),
        scratch_shapes=[
            pltpu.VMEM((M_CHUNK, TN), jnp.float32),
            pltpu.VMEM((N_HOPS, M_CHUNK, TN), jnp.float32),
            pltpu.SemaphoreType.DMA((NTILES, N_HOPS)),
            pltpu.SemaphoreType.DMA((NTILES, N_HOPS)),
        ],
        compiler_params=pltpu.CompilerParams(
            collective_id=0,
            dimension_semantics=("arbitrary",),
            vmem_limit_bytes=62 * 1024 * 1024,
        ),
    )(x, w_mat, scale_x, scale_w)


# device time: 428177 ns/iter; 2.8806x vs baseline; 2.8806x over previous
import jax
import jax.numpy as jnp
from jax import lax
from jax.experimental import pallas as pl
from jax.experimental.pallas import tpu as pltpu

N_DEV = 4
M_CHUNK = 1024
K_SHARD = 1024
N_TOTAL = 8192
TN = 1024
H = TN // 2
NTILES = N_TOTAL // TN
N_HOPS = N_DEV - 1


def kernel(x, w_mat, scale_x, scale_w):
    def body(x_ref, w_ref, sx_ref, sw_ref, out_ref,
             x_bf, w_bf, send_cw, send_ccw, recv_cw, recv_ccw,
             send_sems, recv_sems):
        my = lax.axis_index("i")
        right = (my + 1) % N_DEV
        left = (my + 3) % N_DEV
        nt = pl.program_id(0)

        @pl.when(nt == 0)
        def _():
            x_bf[...] = x_ref[...].astype(jnp.bfloat16)
            barrier_sem = pltpu.get_barrier_semaphore()
            for nbr in [left, right]:
                pl.semaphore_signal(
                    barrier_sem, inc=1,
                    device_id=(nbr,), device_id_type=pl.DeviceIdType.MESH,
                )
            pl.semaphore_wait(barrier_sem, 2)

        w_bf[...] = w_ref[...].astype(jnp.bfloat16)
        scale = sx_ref[0] * sw_ref[0]

        def partial(c, lo, hi):
            xa = x_bf[pl.ds(c * M_CHUNK, M_CHUNK), :]
            return jnp.dot(xa, w_bf[:, lo:hi],
                           preferred_element_type=jnp.float32)

        send_cw[...] = partial((my + 3) % N_DEV, 0, H).astype(jnp.bfloat16)
        send_ccw[...] = partial((my + 1) % N_DEV, H, TN).astype(jnp.bfloat16)
        for s in range(N_HOPS):
            rd_cw = pltpu.make_async_remote_copy(
                src_ref=send_cw,
                dst_ref=recv_cw.at[s],
                send_sem=send_sems.at[nt, s, 0],
                recv_sem=recv_sems.at[nt, s, 0],
                device_id=(right,),
                device_id_type=pl.DeviceIdType.MESH,
            )
            rd_ccw = pltpu.make_async_remote_copy(
                src_ref=send_ccw,
                dst_ref=recv_ccw.at[s],
                send_sem=send_sems.at[nt, s, 1],
                recv_sem=recv_sems.at[nt, s, 1],
                device_id=(left,),
                device_id_type=pl.DeviceIdType.MESH,
            )
            rd_cw.start()
            rd_ccw.start()
            rd_cw.wait()
            rd_ccw.wait()
            if s < N_HOPS - 1:
                send_cw[...] = (
                    recv_cw[s].astype(jnp.float32)
                    + partial((my + 2 - s) % N_DEV, 0, H)
                ).astype(jnp.bfloat16)
                send_ccw[...] = (
                    recv_ccw[s].astype(jnp.float32)
                    + partial((my + 2 + s) % N_DEV, H, TN)
                ).astype(jnp.bfloat16)
            else:
                out_ref[:, 0:H] = (
                    recv_cw[s].astype(jnp.float32) + partial(my, 0, H)
                ) * scale
                out_ref[:, H:TN] = (
                    recv_ccw[s].astype(jnp.float32) + partial(my, H, TN)
                ) * scale

    grid = (NTILES,)
    return pl.pallas_call(
        body,
        grid=grid,
        out_shape=jax.ShapeDtypeStruct((M_CHUNK, N_TOTAL), jnp.float32),
        in_specs=[
            pl.BlockSpec((4096, K_SHARD), lambda nt: (0, 0)),
            pl.BlockSpec((K_SHARD, TN), lambda nt: (0, nt)),
            pl.BlockSpec(memory_space=pltpu.SMEM),
            pl.BlockSpec(memory_space=pltpu.SMEM),
        ],
        out_specs=pl.BlockSpec((M_CHUNK, TN), lambda nt: (0, nt)),
        scratch_shapes=[
            pltpu.VMEM((4096, K_SHARD), jnp.bfloat16),
            pltpu.VMEM((K_SHARD, TN), jnp.bfloat16),
            pltpu.VMEM((M_CHUNK, H), jnp.bfloat16),
            pltpu.VMEM((M_CHUNK, H), jnp.bfloat16),
            pltpu.VMEM((N_HOPS, M_CHUNK, H), jnp.bfloat16),
            pltpu.VMEM((N_HOPS, M_CHUNK, H), jnp.bfloat16),
            pltpu.SemaphoreType.DMA((NTILES, N_HOPS, 2)),
            pltpu.SemaphoreType.DMA((NTILES, N_HOPS, 2)),
        ],
        compiler_params=pltpu.CompilerParams(
            collective_id=0,
            dimension_semantics=("arbitrary",),
            vmem_limit_bytes=62 * 1024 * 1024,
        ),
    )(x, w_mat, scale_x, scale_w)


# device time: 310937 ns/iter; 3.9667x vs baseline; 1.3771x over previous
import jax
import jax.numpy as jnp
from jax import lax
from jax.experimental import pallas as pl
from jax.experimental.pallas import tpu as pltpu

N_DEV = 4
M_CHUNK = 1024
K_SHARD = 1024
N_TOTAL = 8192
TN = 1024
H = TN // 2
NTILES = N_TOTAL // TN
N_HOPS = N_DEV - 1
SEND_SLOTS = 3


def kernel(x, w_mat, scale_x, scale_w):
    def body(x_hbm, w_hbm, sx_ref, sw_ref, out_hbm,
             x_bf, w_bf, stage, out_stage, send_cw, send_ccw,
             recv_cw, recv_ccw, stage_sems, out_sem,
             send_sems, recv_sems, credit_cw, credit_ccw):
        my = lax.axis_index("i")
        right = (my + 1) % N_DEV
        left = (my + 3) % N_DEV

        def dma_in(piece, slot):
            kind, i = piece
            if kind == "x":
                src = x_hbm.at[pl.ds(i * M_CHUNK, M_CHUNK), :]
            else:
                src = w_hbm.at[:, pl.ds(i * TN, TN)]
            return pltpu.make_async_copy(src, stage.at[slot],
                                         stage_sems.at[slot])

        def cast_piece(piece, slot):
            kind, i = piece
            if kind == "x":
                x_bf[pl.ds(i * M_CHUNK, M_CHUNK), :] = (
                    stage[slot].astype(jnp.bfloat16))
            else:
                w_bf[:, i * TN:(i + 1) * TN] = stage[slot].astype(jnp.bfloat16)

        pieces = [("x", i) for i in range(N_DEV)] + \
                 [("w", j) for j in range(NTILES)]
        n_pieces = len(pieces)

        dma_in(pieces[0], 0).start()
        dma_in(pieces[1], 1).start()

        def stage_step(k):
            dma_in(pieces[k], k % 2).wait()
            cast_piece(pieces[k], k % 2)
            if k + 2 < n_pieces:
                dma_in(pieces[k + 2], k % 2).start()

        for k in range(N_DEV):
            stage_step(k)

        barrier_sem = pltpu.get_barrier_semaphore()
        for nbr in [left, right]:
            pl.semaphore_signal(
                barrier_sem, inc=1,
                device_id=(nbr,), device_id_type=pl.DeviceIdType.MESH,
            )
        pl.semaphore_wait(barrier_sem, 2)

        scale = sx_ref[0] * sw_ref[0]

        def partial(c, nt, lo, hi):
            xa = x_bf[pl.ds(c * M_CHUNK, M_CHUNK), :]
            return jnp.dot(xa, w_bf[:, nt * TN + lo:nt * TN + hi],
                           preferred_element_type=jnp.float32)

        def rdma(direction, slot, nt):
            if direction == 0:
                src, dst, dev = send_cw.at[slot], recv_cw.at[nt], (right,)
            else:
                src, dst, dev = send_ccw.at[slot], recv_ccw.at[nt], (left,)
            return pltpu.make_async_remote_copy(
                src_ref=src, dst_ref=dst,
                send_sem=send_sems.at[direction, slot],
                recv_sem=recv_sems.at[direction, nt],
                device_id=dev, device_id_type=pl.DeviceIdType.MESH)

        for p in range(N_HOPS):
            c_cw = (my + 3 - p) % N_DEV
            c_ccw = (my + 1 + p) % N_DEV
            for nt in range(NTILES):
                if p == 0:
                    stage_step(N_DEV + nt)
                k = p * NTILES + nt
                sl = k % SEND_SLOTS
                rd_cw = rdma(0, sl, nt)
                rd_ccw = rdma(1, sl, nt)
                pc_cw = partial(c_cw, nt, 0, H)
                pc_ccw = partial(c_ccw, nt, H, TN)
                if p == 0:
                    acc_cw, acc_ccw = pc_cw, pc_ccw
                else:
                    rd_cw.wait_recv()
                    rd_ccw.wait_recv()
                    acc_cw = pc_cw + recv_cw[nt].astype(jnp.float32)
                    acc_ccw = pc_ccw + recv_ccw[nt].astype(jnp.float32)
                if k >= SEND_SLOTS:
                    rd_cw.wait_send()
                    rd_ccw.wait_send()
                send_cw[sl] = acc_cw.astype(jnp.bfloat16)
                send_ccw[sl] = acc_ccw.astype(jnp.bfloat16)
                if p >= 1:
                    pl.semaphore_signal(
                        credit_cw, inc=1,
                        device_id=(left,),
                        device_id_type=pl.DeviceIdType.MESH)
                    pl.semaphore_signal(
                        credit_ccw, inc=1,
                        device_id=(right,),
                        device_id_type=pl.DeviceIdType.MESH)
                    pl.semaphore_wait(credit_cw, 1)
                    pl.semaphore_wait(credit_ccw, 1)
                rd_cw.start()
                rd_ccw.start()

        for nt in range(NTILES):
            rd_cw = rdma(0, 0, nt)
            rd_ccw = rdma(1, 0, nt)
            rd_cw.wait_recv()
            rd_ccw.wait_recv()
            if nt > 0:
                pltpu.make_async_copy(
                    out_stage, out_hbm.at[:, pl.ds((nt - 1) * TN, TN)],
                    out_sem).wait()
            out_stage[:, 0:H] = (
                partial(my, nt, 0, H) + recv_cw[nt].astype(jnp.float32)
            ) * scale
            out_stage[:, H:TN] = (
                partial(my, nt, H, TN) + recv_ccw[nt].astype(jnp.float32)
            ) * scale
            pltpu.make_async_copy(
                out_stage, out_hbm.at[:, pl.ds(nt * TN, TN)], out_sem).start()
        pltpu.make_async_copy(
            out_stage, out_hbm.at[:, pl.ds((NTILES - 1) * TN, TN)],
            out_sem).wait()

        for d in (0, 1):
            for sl in range(SEND_SLOTS):
                rdma(d, sl, 0).wait_send()

    return pl.pallas_call(
        body,
        out_shape=jax.ShapeDtypeStruct((M_CHUNK, N_TOTAL), jnp.float32),
        in_specs=[
            pl.BlockSpec(memory_space=pl.ANY),
            pl.BlockSpec(memory_space=pl.ANY),
            pl.BlockSpec(memory_space=pltpu.SMEM),
            pl.BlockSpec(memory_space=pltpu.SMEM),
        ],
        out_specs=pl.BlockSpec(memory_space=pl.ANY),
        scratch_shapes=[
            pltpu.VMEM((N_DEV * M_CHUNK, K_SHARD), jnp.bfloat16),
            pltpu.VMEM((K_SHARD, N_TOTAL), jnp.bfloat16),
            pltpu.VMEM((2, M_CHUNK, TN), jnp.float32),
            pltpu.VMEM((M_CHUNK, TN), jnp.float32),
            pltpu.VMEM((SEND_SLOTS, M_CHUNK, H), jnp.bfloat16),
            pltpu.VMEM((SEND_SLOTS, M_CHUNK, H), jnp.bfloat16),
            pltpu.VMEM((NTILES, M_CHUNK, H), jnp.bfloat16),
            pltpu.VMEM((NTILES, M_CHUNK, H), jnp.bfloat16),
            pltpu.SemaphoreType.DMA((2,)),
            pltpu.SemaphoreType.DMA,
            pltpu.SemaphoreType.DMA((2, SEND_SLOTS)),
            pltpu.SemaphoreType.DMA((2, NTILES)),
            pltpu.SemaphoreType.REGULAR,
            pltpu.SemaphoreType.REGULAR,
        ],
        compiler_params=pltpu.CompilerParams(
            collective_id=0,
            vmem_limit_bytes=63 * 1024 * 1024,
        ),
    )(x, w_mat, scale_x, scale_w)
